# baseline (speedup 1.0000x reference)
import jax
import jax.numpy as jnp
from jax import lax
from jax.experimental import pallas as pl
from jax.experimental.pallas import tpu as pltpu
from jax.experimental.pallas import tpu_sc as plsc

B, N, F = 2, 10000, 128
H, C = 4, 32
HID = 128
E = 160000

NC, NS, L = 2, 16, 16
D = 144
K = 64
K2 = 1600
NBC = E // K2
NPT = N // NS
NPTP = 632
WLSZ = K2 + K


def _tc1_body(xf_ref, wc_ref, xe_ref):
    xe_ref[...] = jnp.dot(xf_ref[...], wc_ref[...],
                          preferred_element_type=jnp.float32)


def _tc1(xf, wcomb):
    rb = 800
    return pl.pallas_call(
        _tc1_body,
        grid=(B * N // rb,),
        in_specs=[
            pl.BlockSpec((rb, F), lambda i: (i, 0)),
            pl.BlockSpec((F, D), lambda i: (0, 0)),
        ],
        out_specs=pl.BlockSpec((rb, D), lambda i: (i, 0)),
        out_shape=jax.ShapeDtypeStruct((B * N, D), jnp.float32),
    )(xf, wcomb)


def _sc_body(xe_hbm, src3_hbm, dst3_hbm, sdt_hbm, num_hbm, den_hbm,
             srcv2a, dstv2a, srcv2b, dstv2b, sdl, wl_src, wl_dloc,
             gidxa, gidxb, xja, xjb, wvbuf, acc, accd,
             spfa, spfb, sga, sgb):
    cid = lax.axis_index("c")
    sid = lax.axis_index("s")
    base_off = cid * N
    nlo = sid * NPT
    iota = lax.iota(jnp.int32, L)
    hsel = jnp.minimum(iota, H - 1)
    selv = lax.convert_element_type(
        1 - jnp.minimum(lax.shift_right_logical(iota, 2), 1), jnp.float32)
    zv = jnp.zeros((L,), jnp.float32)
    zi = jnp.zeros((L,), jnp.int32)
    zero16 = iota * 0
    ibufs = ((srcv2a, dstv2a, spfa), (srcv2b, dstv2b, spfb))
    gbufs = ((gidxa, xja, sga), (gidxb, xjb, sgb))

    pltpu.sync_copy(sdt_hbm.at[cid, sid], sdl)

    def zrow(r, c):
        acc[pl.ds(r * L, L)] = zv
        return c

    lax.fori_loop(0, NPTP * 128 // L, zrow, 0)

    def zrowd(r, c):
        accd[pl.ds(r * L, L)] = zv
        return c

    lax.fori_loop(0, NPTP * 16 // L, zrowd, 0)

    pltpu.async_copy(src3_hbm.at[0], srcv2a, spfa)
    pltpu.async_copy(dst3_hbm.at[0], dstv2a, spfa)

    def build_issue(bI, qb):
        gI, xI, sI = gbufs[qb]
        for g in range(K // L):
            gI[pl.ds(g * L, L)] = wl_src[pl.ds(bI * K + g * L, L)] + base_off
        pltpu.async_copy(xe_hbm.at[gI], xI, sI)

    def process_chunk(c, pb):
        srcv2, dstv2, spf = ibufs[pb]
        pltpu.make_async_copy(src3_hbm.at[c], srcv2, spf).wait()
        pltpu.make_async_copy(dst3_hbm.at[c], dstv2, spf).wait()

        @pl.when(c + 1 < NBC)
        def _():
            nsrc, ndst, nspf = ibufs[1 - pb]
            pltpu.async_copy(src3_hbm.at[c + 1], nsrc, nspf)
            pltpu.async_copy(dst3_hbm.at[c + 1], ndst, nspf)

        def scan_g(g, off):
            sv = srcv2[0, pl.ds(g * L, L)]
            dv = dstv2[0, pl.ds(g * L, L)]
            dl = dv - nlo
            tt = jnp.bitwise_or(dl, (NPT - 1) - dl)
            keep = tt >= 0
            plsc.store_compressed(wl_src.at[pl.ds(off, L)], sv, mask=keep)
            plsc.store_compressed(wl_dloc.at[pl.ds(off, L)], dl, mask=keep)
            cnt = plsc.all_reduce_population_count(keep)[0]
            return off + cnt

        nkept = lax.fori_loop(0, K2 // L, scan_g, 0)

        for b in range(K // L):
            wl_src[pl.ds(nkept + b * L, L)] = zi
            wl_dloc[pl.ds(nkept + b * L, L)] = jnp.full((L,), NPT, jnp.int32)

        nblk = (nkept + K - 1) // K

        @pl.when(nblk > 0)
        def _():
            build_issue(0, 0)

        def process_blk(bI, qb):
            gI, xI, sI = gbufs[qb]
            pltpu.make_async_copy(xe_hbm.at[gI], xI, sI).wait()

            @pl.when(bI + 1 < nblk)
            def _():
                build_issue(bI + 1, 1 - qb)

            bo = bI * K

            def grp(g, c3):
                eb = g * L
                dlv = wl_dloc[pl.ds(bo + eb, L)]
                for l in range(L):
                    e = eb + l
                    d_e = dlv[l]
                    dlb = zero16 + d_e
                    v128 = xI[e, pl.ds(128, L)]
                    sdv = plsc.load_gather(sdl, [dlb * H + hsel])
                    av = v128 + sdv
                    av = jnp.maximum(av, 0.2 * av)
                    wv = jnp.exp(av) * selv
                    plsc.addupdate_scatter(accd, [dlb * 16 + iota], wv)
                    base = dlb * 128 + iota
                    for h in range(H):
                        wh = wv[h]
                        for u in range(C // L):
                            j = (h * (C // L) + u) * L
                            plsc.addupdate_scatter(
                                acc, [base + j], xI[e, pl.ds(j, L)] * wh)
                return c3

            lax.fori_loop(0, K // L, grp, 0)

        def blkpair(bp, c2):
            for qb in (0, 1):
                bI = bp * 2 + qb

                @pl.when(bI < nblk)
                def _():
                    process_blk(bI, qb)
            return c2

        lax.fori_loop(0, (nblk + 1) // 2, blkpair, 0)

    def chunkpair(p, carry):
        for pb in (0, 1):
            process_chunk(p * 2 + pb, pb)
        return carry

    lax.fori_loop(0, NBC // 2, chunkpair, 0)

    pltpu.sync_copy(acc, num_hbm.at[cid, sid])
    pltpu.sync_copy(accd, den_hbm.at[cid, sid])


def _sc(xe, src3, dst3, sdt):
    mesh = plsc.VectorSubcoreMesh(core_axis_name="c", subcore_axis_name="s")
    return pl.kernel(
        _sc_body,
        out_type=[
            jax.ShapeDtypeStruct((B, NS, NPTP * 128), jnp.float32),
            jax.ShapeDtypeStruct((B, NS, NPTP * 16), jnp.float32),
        ],
        mesh=mesh,
        compiler_params=pltpu.CompilerParams(needs_layout_passes=False,
                                             use_tc_tiling_on_sc=False),
        scratch_types=[
            pltpu.VMEM((1, K2), jnp.int32),
            pltpu.VMEM((1, K2), jnp.int32),
            pltpu.VMEM((1, K2), jnp.int32),
            pltpu.VMEM((1, K2), jnp.int32),
            pltpu.VMEM((NPTP * H,), jnp.float32),
            pltpu.VMEM((WLSZ,), jnp.int32),
            pltpu.VMEM((WLSZ,), jnp.int32),
            pltpu.VMEM((K,), jnp.int32),
            pltpu.VMEM((K,), jnp.int32),
            pltpu.VMEM((K, D), jnp.float32),
            pltpu.VMEM((K, D), jnp.float32),
            pltpu.VMEM((L,), jnp.float32),
            pltpu.VMEM((NPTP * 128,), jnp.float32),
            pltpu.VMEM((NPTP * 16,), jnp.float32),
            pltpu.SemaphoreType.DMA,
            pltpu.SemaphoreType.DMA,
            pltpu.SemaphoreType.DMA,
            pltpu.SemaphoreType.DMA,
        ],
    )(xe, src3, dst3, sdt)


def _tc2_body(xe_ref, num_ref, den_ref, am_ref, md_ref, bias_ref, w3_ref,
              b3_ref, gamma_ref, beta_ref, h_ref, c_ref):
    xe = xe_ref[...]
    xt = xe[:, 0:128]
    zb = jnp.dot(xe, am_ref[...], preferred_element_type=jnp.float32)
    wb = jnp.exp(jnp.maximum(zb, 0.2 * zb))
    num = num_ref[...] + wb * xt
    den = (jnp.dot(den_ref[...], md_ref[...], preferred_element_type=jnp.float32)
           + wb)
    g = num / (den + 1e-16) + bias_ref[...]
    gates = jnp.dot(g, w3_ref[...], preferred_element_type=jnp.float32) + b3_ref[...]
    i_g = jax.nn.sigmoid(gates[:, 0:HID])
    g_g = jnp.tanh(gates[:, HID:2 * HID])
    o_g = jax.nn.sigmoid(gates[:, 2 * HID:3 * HID])
    c = i_g * g_g
    hh = o_g * jnp.tanh(c)
    mu = jnp.mean(hh, axis=1, keepdims=True)
    var = jnp.mean((hh - mu) ** 2, axis=1, keepdims=True)
    h_ref[...] = (hh - mu) * lax.rsqrt(var + 1e-5) * gamma_ref[...] + beta_ref[...]
    c_ref[...] = c


def _tc2(xe, num, den, am, md, bias, w3, b3, gamma, beta):
    rb = 800
    return pl.pallas_call(
        _tc2_body,
        grid=(B * N // rb,),
        in_specs=[
            pl.BlockSpec((rb, D), lambda i: (i, 0)),
            pl.BlockSpec((rb, 128), lambda i: (i, 0)),
            pl.BlockSpec((rb, 16), lambda i: (i, 0)),
            pl.BlockSpec((D, 128), lambda i: (0, 0)),
            pl.BlockSpec((16, 128), lambda i: (0, 0)),
            pl.BlockSpec((1, 128), lambda i: (0, 0)),
            pl.BlockSpec((128, 384), lambda i: (0, 0)),
            pl.BlockSpec((1, 384), lambda i: (0, 0)),
            pl.BlockSpec((1, 128), lambda i: (0, 0)),
            pl.BlockSpec((1, 128), lambda i: (0, 0)),
        ],
        out_specs=[
            pl.BlockSpec((rb, HID), lambda i: (i, 0)),
            pl.BlockSpec((rb, HID), lambda i: (i, 0)),
        ],
        out_shape=[
            jax.ShapeDtypeStruct((B * N, HID), jnp.float32),
            jax.ShapeDtypeStruct((B * N, HID), jnp.float32),
        ],
    )(xe, num, den, am, md, bias, w3, b3, gamma, beta)


@jax.jit
def kernel(x, edge_index, W_lin, att_src, att_dst, bias, W_ih, W_hh,
           b_ih, b_hh, gamma, beta):
    xf = x.reshape(B * N, F)
    wlt = W_lin.T

    mh = jnp.repeat(jnp.eye(H, dtype=jnp.float32), C, axis=0)
    a1 = jnp.concatenate(
        [mh * att_src.reshape(H * C)[:, None],
         mh * att_dst.reshape(H * C)[:, None]], axis=1)
    wcomb = jnp.concatenate(
        [wlt, wlt @ a1, jnp.zeros((F, D - F - 8), jnp.float32)], axis=1)

    xe = _tc1(xf, wcomb)

    src3 = edge_index[0].reshape(NBC, 1, K2)
    dst3 = edge_index[1].reshape(NBC, 1, K2)
    sdst = xe[:, 132:136].reshape(B, NS, NPT, H)
    sdt = jnp.concatenate(
        [sdst, jnp.full((B, NS, NPTP - NPT, H), -40.0, jnp.float32)],
        axis=2).reshape(B, NS, NPTP * H)

    num, den = _sc(xe, src3, dst3, sdt)
    numf = num.reshape(B, NS, NPTP, 128)[:, :, :NPT, :].reshape(B * N, 128)
    denf = den.reshape(B, NS, NPTP, 16)[:, :, :NPT, :].reshape(B * N, 16)

    mht = mh.T
    am = jnp.zeros((D, 128), jnp.float32)
    am = am.at[128:132].set(mht).at[132:136].set(mht)
    md = jnp.concatenate([mht, jnp.zeros((12, 128), jnp.float32)], axis=0)

    w_ihT = W_ih.T
    w3 = jnp.concatenate([w_ihT[:, 0:HID], w_ihT[:, 2 * HID:]], axis=1)
    bsum = b_ih + b_hh
    b3 = jnp.concatenate([bsum[0:HID], bsum[2 * HID:]]).reshape(1, 3 * HID)

    h, c = _tc2(xe, numf, denf, am, md, bias.reshape(1, HID), w3, b3,
                gamma.reshape(1, HID), beta.reshape(1, HID))
    return h.reshape(B, N, HID), c.reshape(B, N, HID)


# speedup vs baseline: 1.6722x; 1.6722x over previous
import jax
import jax.numpy as jnp
from jax import lax
from jax.experimental import pallas as pl
from jax.experimental.pallas import tpu as pltpu
from jax.experimental.pallas import tpu_sc as plsc

B, N, F = 2, 10000, 128
H, C = 4, 32
HID = 128
E = 160000

NC, NS, L = 2, 16, 16
D = 144
K = 64
K2 = 1600
NBC = E // K2
NPT = N // NS
NPTP = 632
WLSZ = K2 + K


def _tc1_body(xf_ref, wc_ref, xe_ref):
    xe_ref[...] = jnp.dot(xf_ref[...], wc_ref[...],
                          preferred_element_type=jnp.float32)


def _tc1(xf, wcomb):
    rb = 800
    return pl.pallas_call(
        _tc1_body,
        grid=(B * N // rb,),
        in_specs=[
            pl.BlockSpec((rb, F), lambda i: (i, 0)),
            pl.BlockSpec((F, D), lambda i: (0, 0)),
        ],
        out_specs=pl.BlockSpec((rb, D), lambda i: (i, 0)),
        out_shape=jax.ShapeDtypeStruct((B * N, D), jnp.float32),
    )(xf, wcomb)


def _sc_body(xe_hbm, src3_hbm, dst3_hbm, sdt_hbm, nd_hbm,
             srcv2a, dstv2a, srcv2b, dstv2b, sdl, wl_src, wl_dloc,
             gidxa, gidxb, xja, xjb,
             a0, a1, a2, a3, a4, a5, a6, a7, accd,
             spfa, spfb, sga, sgb):
    accs = (a0, a1, a2, a3, a4, a5, a6, a7)
    cid = lax.axis_index("c")
    sid = lax.axis_index("s")
    base_off = cid * N
    nlo = sid * NPT
    iota = lax.iota(jnp.int32, L)
    hsel = jnp.minimum(iota, H - 1)
    selv = lax.convert_element_type(
        1 - jnp.minimum(lax.shift_right_logical(iota, 2), 1), jnp.float32)
    zv = jnp.zeros((L,), jnp.float32)
    zi = jnp.zeros((L,), jnp.int32)
    zero16 = iota * 0
    ibufs = ((srcv2a, dstv2a, spfa), (srcv2b, dstv2b, spfb))
    gbufs = ((gidxa, xja, sga), (gidxb, xjb, sgb))

    pltpu.sync_copy(sdt_hbm.at[cid, sid], sdl)

    def zrow(r, c):
        for a in accs:
            a[pl.ds(r * L, L)] = zv
        accd[pl.ds(r * L, L)] = zv
        return c

    lax.fori_loop(0, NPTP * 16 // L, zrow, 0)

    pltpu.async_copy(src3_hbm.at[0], srcv2a, spfa)
    pltpu.async_copy(dst3_hbm.at[0], dstv2a, spfa)

    def build_issue(bI, qb):
        gI, xI, sI = gbufs[qb]
        for g in range(K // L):
            gI[pl.ds(g * L, L)] = wl_src[pl.ds(bI * K + g * L, L)] + base_off
        pltpu.async_copy(xe_hbm.at[gI], xI, sI)

    def process_chunk(c, pb):
        srcv2, dstv2, spf = ibufs[pb]
        pltpu.make_async_copy(src3_hbm.at[c], srcv2, spf).wait()
        pltpu.make_async_copy(dst3_hbm.at[c], dstv2, spf).wait()

        @pl.when(c + 1 < NBC)
        def _():
            nsrc, ndst, nspf = ibufs[1 - pb]
            pltpu.async_copy(src3_hbm.at[c + 1], nsrc, nspf)
            pltpu.async_copy(dst3_hbm.at[c + 1], ndst, nspf)

        def scan_g(g, off):
            sv = srcv2[0, pl.ds(g * L, L)]
            dv = dstv2[0, pl.ds(g * L, L)]
            dl = dv - nlo
            tt = jnp.bitwise_or(dl, (NPT - 1) - dl)
            keep = tt >= 0
            plsc.store_compressed(wl_src.at[pl.ds(off, L)], sv, mask=keep)
            plsc.store_compressed(wl_dloc.at[pl.ds(off, L)], dl, mask=keep)
            cnt = plsc.all_reduce_population_count(keep)[0]
            return off + cnt

        nkept = lax.fori_loop(0, K2 // L, scan_g, 0)

        for b in range(K // L):
            wl_src[pl.ds(nkept + b * L, L)] = zi
            wl_dloc[pl.ds(nkept + b * L, L)] = jnp.full((L,), NPT, jnp.int32)

        nblk = (nkept + K - 1) // K

        @pl.when(nblk > 0)
        def _():
            build_issue(0, 0)

        def process_blk(bI, qb):
            gI, xI, sI = gbufs[qb]
            pltpu.make_async_copy(xe_hbm.at[gI], xI, sI).wait()

            @pl.when(bI + 1 < nblk)
            def _():
                build_issue(bI + 1, 1 - qb)

            bo = bI * K

            def grp(g, c3):
                eb = g * L
                dlv = wl_dloc[pl.ds(bo + eb, L)]
                for l in range(L):
                    e = eb + l
                    d_e = dlv[l]
                    dlb = zero16 + d_e
                    v128 = xI[e, pl.ds(128, L)]
                    sdv = plsc.load_gather(sdl, [dlb * H + hsel])
                    av = v128 + sdv
                    av = jnp.maximum(av, 0.2 * av)
                    wv = jnp.exp(av) * selv
                    idx16 = dlb * L + iota
                    plsc.addupdate_scatter(accd, [idx16], wv)
                    for h in range(H):
                        wh = wv[h]
                        for u2 in range(C // L):
                            u = h * (C // L) + u2
                            plsc.addupdate_scatter(
                                accs[u], [idx16], xI[e, pl.ds(u * L, L)] * wh)
                return c3

            lax.fori_loop(0, K // L, grp, 0)

        def blkpair(bp, c2):
            for qb in (0, 1):
                bI = bp * 2 + qb

                @pl.when(bI < nblk)
                def _():
                    process_blk(bI, qb)
            return c2

        lax.fori_loop(0, (nblk + 1) // 2, blkpair, 0)

    def chunkpair(p, carry):
        for pb in (0, 1):
            process_chunk(p * 2 + pb, pb)
        return carry

    lax.fori_loop(0, NBC // 2, chunkpair, 0)

    for u in range(8):
        pltpu.sync_copy(accs[u], nd_hbm.at[cid, sid, u])
    pltpu.sync_copy(accd, nd_hbm.at[cid, sid, 8])


def _sc(xe, src3, dst3, sdt):
    mesh = plsc.VectorSubcoreMesh(core_axis_name="c", subcore_axis_name="s")
    return pl.kernel(
        _sc_body,
        out_type=jax.ShapeDtypeStruct((B, NS, 9, NPTP * 16), jnp.float32),
        mesh=mesh,
        compiler_params=pltpu.CompilerParams(needs_layout_passes=False,
                                             use_tc_tiling_on_sc=False),
        scratch_types=[
            pltpu.VMEM((1, K2), jnp.int32),
            pltpu.VMEM((1, K2), jnp.int32),
            pltpu.VMEM((1, K2), jnp.int32),
            pltpu.VMEM((1, K2), jnp.int32),
            pltpu.VMEM((NPTP * H,), jnp.float32),
            pltpu.VMEM((WLSZ,), jnp.int32),
            pltpu.VMEM((WLSZ,), jnp.int32),
            pltpu.VMEM((K,), jnp.int32),
            pltpu.VMEM((K,), jnp.int32),
            pltpu.VMEM((K, D), jnp.float32),
            pltpu.VMEM((K, D), jnp.float32),
        ] + [pltpu.VMEM((NPTP * 16,), jnp.float32)] * 9 + [
            pltpu.SemaphoreType.DMA,
            pltpu.SemaphoreType.DMA,
            pltpu.SemaphoreType.DMA,
            pltpu.SemaphoreType.DMA,
        ],
    )(xe, src3, dst3, sdt)


def _tc2_body(xe_ref, num_ref, den_ref, am_ref, md_ref, bias_ref, w3_ref,
              b3_ref, gamma_ref, beta_ref, h_ref, c_ref):
    xe = xe_ref[...]
    xt = xe[:, 0:128]
    zb = jnp.dot(xe, am_ref[...], preferred_element_type=jnp.float32)
    wb = jnp.exp(jnp.maximum(zb, 0.2 * zb))
    num = num_ref[...] + wb * xt
    den = (jnp.dot(den_ref[...], md_ref[...], preferred_element_type=jnp.float32)
           + wb)
    g = num / (den + 1e-16) + bias_ref[...]
    gates = jnp.dot(g, w3_ref[...], preferred_element_type=jnp.float32) + b3_ref[...]
    i_g = jax.nn.sigmoid(gates[:, 0:HID])
    g_g = jnp.tanh(gates[:, HID:2 * HID])
    o_g = jax.nn.sigmoid(gates[:, 2 * HID:3 * HID])
    c = i_g * g_g
    hh = o_g * jnp.tanh(c)
    mu = jnp.mean(hh, axis=1, keepdims=True)
    var = jnp.mean((hh - mu) ** 2, axis=1, keepdims=True)
    h_ref[...] = (hh - mu) * lax.rsqrt(var + 1e-5) * gamma_ref[...] + beta_ref[...]
    c_ref[...] = c


def _tc2(xe, num, den, am, md, bias, w3, b3, gamma, beta):
    rb = 800
    return pl.pallas_call(
        _tc2_body,
        grid=(B * N // rb,),
        in_specs=[
            pl.BlockSpec((rb, D), lambda i: (i, 0)),
            pl.BlockSpec((rb, 128), lambda i: (i, 0)),
            pl.BlockSpec((rb, 16), lambda i: (i, 0)),
            pl.BlockSpec((D, 128), lambda i: (0, 0)),
            pl.BlockSpec((16, 128), lambda i: (0, 0)),
            pl.BlockSpec((1, 128), lambda i: (0, 0)),
            pl.BlockSpec((128, 384), lambda i: (0, 0)),
            pl.BlockSpec((1, 384), lambda i: (0, 0)),
            pl.BlockSpec((1, 128), lambda i: (0, 0)),
            pl.BlockSpec((1, 128), lambda i: (0, 0)),
        ],
        out_specs=[
            pl.BlockSpec((rb, HID), lambda i: (i, 0)),
            pl.BlockSpec((rb, HID), lambda i: (i, 0)),
        ],
        out_shape=[
            jax.ShapeDtypeStruct((B * N, HID), jnp.float32),
            jax.ShapeDtypeStruct((B * N, HID), jnp.float32),
        ],
    )(xe, num, den, am, md, bias, w3, b3, gamma, beta)


@jax.jit
def kernel(x, edge_index, W_lin, att_src, att_dst, bias, W_ih, W_hh,
           b_ih, b_hh, gamma, beta):
    xf = x.reshape(B * N, F)
    wlt = W_lin.T

    mh = jnp.repeat(jnp.eye(H, dtype=jnp.float32), C, axis=0)
    a1 = jnp.concatenate(
        [mh * att_src.reshape(H * C)[:, None],
         mh * att_dst.reshape(H * C)[:, None]], axis=1)
    wcomb = jnp.concatenate(
        [wlt, wlt @ a1, jnp.zeros((F, D - F - 8), jnp.float32)], axis=1)

    xe = _tc1(xf, wcomb)

    src3 = edge_index[0].reshape(NBC, 1, K2)
    dst3 = edge_index[1].reshape(NBC, 1, K2)
    sdst = xe[:, 132:136].reshape(B, NS, NPT, H)
    sdt = jnp.concatenate(
        [sdst, jnp.full((B, NS, NPTP - NPT, H), -40.0, jnp.float32)],
        axis=2).reshape(B, NS, NPTP * H)

    nd = _sc(xe, src3, dst3, sdt)
    nd = nd.reshape(B, NS, 9, NPTP, 16).transpose(0, 1, 3, 2, 4)
    numf = nd[:, :, :NPT, 0:8, :].reshape(B * N, 128)
    denf = nd[:, :, :NPT, 8, :].reshape(B * N, 16)

    mht = mh.T
    am = jnp.zeros((D, 128), jnp.float32)
    am = am.at[128:132].set(mht).at[132:136].set(mht)
    md = jnp.concatenate([mht, jnp.zeros((12, 128), jnp.float32)], axis=0)

    w_ihT = W_ih.T
    w3 = jnp.concatenate([w_ihT[:, 0:HID], w_ihT[:, 2 * HID:]], axis=1)
    bsum = b_ih + b_hh
    b3 = jnp.concatenate([bsum[0:HID], bsum[2 * HID:]]).reshape(1, 3 * HID)

    h, c = _tc2(xe, numf, denf, am, md, bias.reshape(1, HID), w3, b3,
                gamma.reshape(1, HID), beta.reshape(1, HID))
    return h.reshape(B, N, HID), c.reshape(B, N, HID)
